# baseline (speedup 1.0000x reference)
import jax
import jax.numpy as jnp
from jax.experimental import pallas as pl
from jax.experimental.pallas import tpu as pltpu

_N = 32
_C = 192
_HW = 56
_IMGN = _C * _HW * _HW
_COUNT_M = float(_N * _IMGN)
_INF = 1 << 24

_ASHAPE = (_HW, _HW, _C)


def _threefry_bits(x2):
    ks0 = jnp.uint32(0)
    ks1 = jnp.uint32(42)
    ks2 = ks0 ^ ks1 ^ jnp.uint32(0x1BD11BDA)

    def rnds(v0, v1, rots):
        for r in rots:
            v0 = v0 + v1
            v1 = (v1 << r) | (v1 >> (32 - r))
            v1 = v0 ^ v1
        return v0, v1

    w = x2 + ks1
    v0 = w
    v1 = w ^ ((w << 13) | (w >> 19))
    v0, v1 = rnds(v0, v1, (15, 26, 6))
    v0 = v0 + ks1
    v1 = v1 + (ks2 + jnp.uint32(1))
    v0, v1 = rnds(v0, v1, (17, 29, 16, 24))
    v0 = v0 + ks2
    v1 = v1 + (ks0 + jnp.uint32(2))
    v0, v1 = rnds(v0, v1, (13, 15, 26, 6))
    v0 = v0 + ks0
    v1 = v1 + (ks1 + jnp.uint32(3))
    v0, v1 = rnds(v0, v1, (17, 29, 16, 24))
    v0 = v0 + ks1
    v1 = v1 + (ks2 + jnp.uint32(4))
    v0, v1 = rnds(v0, v1, (13, 15, 26, 6))
    v0 = v0 + ks2
    v1 = v1 + (ks0 + jnp.uint32(5))
    return v0 ^ v1


_CP = 2 * _C


def _mask_kernel(gf_ref, packed_ref, count_ref, rbuf, s2buf, pk0, pk1, acc):
    i = pl.program_id(0)
    base = (i * (2 * _IMGN)).astype(jnp.uint32)
    gf = gf_ref[0, 0]
    inf = jnp.float32(_INF)

    rshape = (_HW, _CP)
    cq = jax.lax.broadcasted_iota(jnp.uint32, rshape, 0)
    cw = jax.lax.broadcasted_iota(jnp.uint32, rshape, 1)
    idx0 = base + cw * jnp.uint32(_HW * _HW) + cq
    qiota = cq.astype(jnp.int32)
    qm1 = qiota < 1
    qm2 = qiota < 2
    qm3 = qiota < 3

    pk0[...] = jnp.zeros(rshape, jnp.int32)
    pk1[...] = jnp.zeros(rshape, jnp.int32)
    acc[...] = jnp.zeros(rshape, jnp.int32)

    def row_min(p_idx):
        bits = _threefry_bits(idx0 + (p_idx * _HW).astype(jnp.uint32))
        m = (bits >> 9).astype(jnp.int32).astype(jnp.float32)
        t = jnp.minimum(m, jnp.where(qm1, inf, jnp.roll(m, 1, axis=0)))
        t = jnp.minimum(t, jnp.where(qm2, inf, jnp.roll(t, 2, axis=0)))
        return jnp.minimum(t, jnp.where(qm3, inf, jnp.roll(t, 3, axis=0)))

    def emit(p_idx, mu, static_plane=None):
        keep = (mu >= gf).astype(jnp.int32)
        acc[...] += keep
        contrib = keep << (p_idx & 31)
        if static_plane is not None:
            static_plane[...] |= contrib
        else:
            @pl.when(p_idx < 32)
            def _():
                pk0[...] |= contrib

            @pl.when(p_idx >= 32)
            def _():
                pk1[...] |= contrib

    rm = None
    for p in range(6):
        r = row_min(jnp.int32(p))
        rbuf[p] = r
        if p >= 1:
            s2buf[p] = jnp.minimum(r, rbuf[p - 1])
        rm = r if rm is None else jnp.minimum(rm, r)
        emit(p, rm, static_plane=pk0)

    def body(p, carry):
        r = row_min(p)
        s2 = jnp.minimum(r, rbuf[(p - 1) & 7])
        s4 = jnp.minimum(s2, s2buf[(p - 2) & 7])
        mu = jnp.minimum(jnp.minimum(s4, s2buf[(p - 4) & 7]), rbuf[(p - 6) & 7])
        rbuf[p & 7] = r
        s2buf[p & 7] = s2
        emit(p, mu)
        return carry

    jax.lax.fori_loop(6, _HW, body, 0)

    packed_ref[0, 0, 0] = pk0[:, 0:_C]
    packed_ref[0, 0, 1] = pk1[:, 0:_C]
    packed_ref[0, 1, 0] = pk0[:, _C:_CP]
    packed_ref[0, 1, 1] = pk1[:, _C:_CP]

    @pl.when(i == 0)
    def _():
        count_ref[0, 0] = 0

    count_ref[0, 0] += jnp.sum(acc[...])


def _scale_kernel(count_ref, x_ref, packed_ref, out_ref):
    scale = jnp.float32(_COUNT_M) / count_ref[0, 0].astype(jnp.float32)
    pa = packed_ref[0, 0]
    pb = packed_ref[0, 1]
    piota = jax.lax.broadcasted_iota(jnp.int32, _ASHAPE, 0)
    src = jnp.where(piota < 32, pa[None], pb[None])
    sh = jnp.where(piota < 32, piota, piota - 32)
    bits = (src >> sh) & 1
    out_ref[0] = x_ref[0] * (bits.astype(jnp.float32) * scale)


def kernel(x, gamma):
    xt = jnp.transpose(x, (0, 2, 3, 1))
    gf = jnp.ceil(gamma * jnp.float32(8388608.0)).reshape(1, 1)

    packed, count = pl.pallas_call(
        _mask_kernel,
        grid=(_N // 2,),
        in_specs=[pl.BlockSpec(memory_space=pltpu.SMEM)],
        out_specs=[
            pl.BlockSpec((1, 2, 2, _HW, _C), lambda i: (i, 0, 0, 0, 0)),
            pl.BlockSpec(memory_space=pltpu.SMEM),
        ],
        out_shape=[
            jax.ShapeDtypeStruct((_N // 2, 2, 2, _HW, _C), jnp.int32),
            jax.ShapeDtypeStruct((1, 1), jnp.int32),
        ],
        scratch_shapes=[
            pltpu.VMEM((8, _HW, _CP), jnp.float32),
            pltpu.VMEM((8, _HW, _CP), jnp.float32),
            pltpu.VMEM((_HW, _CP), jnp.int32),
            pltpu.VMEM((_HW, _CP), jnp.int32),
            pltpu.VMEM((_HW, _CP), jnp.int32),
        ],
    )(gf)
    packed = packed.reshape(_N, 2, _HW, _C)

    out = pl.pallas_call(
        _scale_kernel,
        grid=(_N,),
        in_specs=[
            pl.BlockSpec(memory_space=pltpu.SMEM),
            pl.BlockSpec((1, _HW, _HW, _C), lambda n: (n, 0, 0, 0)),
            pl.BlockSpec((1, 2, _HW, _C), lambda n: (n, 0, 0, 0)),
        ],
        out_specs=pl.BlockSpec((1, _HW, _HW, _C), lambda n: (n, 0, 0, 0)),
        out_shape=jax.ShapeDtypeStruct((_N, _HW, _HW, _C), jnp.float32),
    )(count, xt, packed)

    return jnp.transpose(out, (0, 3, 1, 2))
